# Initial kernel scaffold; baseline (speedup 1.0000x reference)
import jax
import jax.numpy as jnp
from jax.experimental import pallas as pl


def kernel(x, adj, W0, b0, W1, b1):
    raise NotImplementedError("write your pallas kernel here")


# speedup vs baseline: 1.0400x; 1.0400x over previous
import functools

import jax
import jax.numpy as jnp
from jax.experimental import pallas as pl
from jax.experimental.pallas import tpu as pltpu


def _layer1_body(x_ref, w0_ref, b0_ref, adj_ref, h_ref, stats_ref, y0_ref):
    i = pl.program_id(0)

    @pl.when(i == 0)
    def _init():
        y0_ref[...] = jnp.dot(x_ref[...], w0_ref[...],
                              preferred_element_type=jnp.float32)
        stats_ref[...] = jnp.zeros_like(stats_ref)

    h = jnp.dot(adj_ref[...], y0_ref[...],
                preferred_element_type=jnp.float32) + b0_ref[...]
    h = jnp.maximum(h, 0.0)
    h_ref[...] = h
    stats_ref[0:1, :] += jnp.sum(h, axis=0, keepdims=True)
    stats_ref[1:2, :] += jnp.sum(h * h, axis=0, keepdims=True)


def _layer2_body(h_ref, w1_ref, mean_ref, scale_ref, b1_ref, adj_ref,
                 out_ref, y1_ref):
    i = pl.program_id(0)

    @pl.when(i == 0)
    def _init():
        hn = (h_ref[...] - mean_ref[...]) * scale_ref[...]
        y1_ref[...] = jnp.dot(hn, w1_ref[...],
                              preferred_element_type=jnp.float32)

    out_ref[...] = jnp.dot(adj_ref[...], y1_ref[...],
                           preferred_element_type=jnp.float32) + b1_ref[...]


@functools.partial(jax.jit, static_argnames=("block_m",))
def _gcn(x, adj, W0, b0, W1, b1, block_m=400):
    n, d = x.shape
    h_dim = W0.shape[1]
    o_dim = W1.shape[1]
    grid = (n // block_m,)

    b0r = b0.reshape(1, h_dim)
    b1r = b1.reshape(1, o_dim)

    h, stats = pl.pallas_call(
        _layer1_body,
        grid=grid,
        in_specs=[
            pl.BlockSpec((n, d), lambda i: (0, 0)),
            pl.BlockSpec((d, h_dim), lambda i: (0, 0)),
            pl.BlockSpec((1, h_dim), lambda i: (0, 0)),
            pl.BlockSpec((block_m, n), lambda i: (i, 0)),
        ],
        out_specs=[
            pl.BlockSpec((block_m, h_dim), lambda i: (i, 0)),
            pl.BlockSpec((8, h_dim), lambda i: (0, 0)),
        ],
        out_shape=[
            jax.ShapeDtypeStruct((n, h_dim), jnp.float32),
            jax.ShapeDtypeStruct((8, h_dim), jnp.float32),
        ],
        scratch_shapes=[pltpu.VMEM((n, h_dim), jnp.float32)],
    )(x, W0, b0r, adj)

    mean = stats[0:1, :] / n
    var = stats[1:2, :] / n - mean * mean
    scale = jax.lax.rsqrt(var + 1e-5)

    out = pl.pallas_call(
        _layer2_body,
        grid=grid,
        in_specs=[
            pl.BlockSpec((n, h_dim), lambda i: (0, 0)),
            pl.BlockSpec((h_dim, o_dim), lambda i: (0, 0)),
            pl.BlockSpec((1, h_dim), lambda i: (0, 0)),
            pl.BlockSpec((1, h_dim), lambda i: (0, 0)),
            pl.BlockSpec((1, o_dim), lambda i: (0, 0)),
            pl.BlockSpec((block_m, n), lambda i: (i, 0)),
        ],
        out_specs=pl.BlockSpec((block_m, o_dim), lambda i: (i, 0)),
        out_shape=jax.ShapeDtypeStruct((n, o_dim), jnp.float32),
        scratch_shapes=[pltpu.VMEM((n, o_dim), jnp.float32)],
    )(h, W1, mean, scale, b1r, adj)
    return out


def kernel(x, adj, W0, b0, W1, b1):
    return _gcn(x, adj, W0, b0, W1, b1)


# speedup vs baseline: 1.0766x; 1.0353x over previous
import functools

import jax
import jax.numpy as jnp
from jax.experimental import pallas as pl
from jax.experimental.pallas import tpu as pltpu


def _gcn_body(x_ref, w0_ref, b0_ref, w1_ref, b1_ref, adj_ref, out_ref,
              y0_ref, h_ref, y1_ref, stats_ref, *, nb, block_m, n):
    i = pl.program_id(0)

    @pl.when(i == 0)
    def _init():
        y0_ref[...] = jnp.dot(x_ref[...], w0_ref[...],
                              preferred_element_type=jnp.float32)
        stats_ref[...] = jnp.zeros_like(stats_ref)

    @pl.when(i < nb)
    def _layer1():
        h = jnp.dot(adj_ref[...], y0_ref[...],
                    preferred_element_type=jnp.float32) + b0_ref[...]
        h = jnp.maximum(h, 0.0)
        h_ref[pl.ds(i * block_m, block_m), :] = h
        stats_ref[0:1, :] += jnp.sum(h, axis=0, keepdims=True)
        stats_ref[1:2, :] += jnp.sum(h * h, axis=0, keepdims=True)

    @pl.when(i == nb)
    def _bn_project():
        mean = stats_ref[0:1, :] / n
        var = stats_ref[1:2, :] / n - mean * mean
        scale = jax.lax.rsqrt(var + 1e-5)
        hn = (h_ref[...] - mean) * scale
        y1_ref[...] = jnp.dot(hn, w1_ref[...],
                              preferred_element_type=jnp.float32)

    @pl.when(i >= nb)
    def _layer2():
        out_ref[...] = jnp.dot(adj_ref[...], y1_ref[...],
                               preferred_element_type=jnp.float32) + b1_ref[...]


@functools.partial(jax.jit, static_argnames=("block_m",))
def _gcn(x, adj, W0, b0, W1, b1, block_m=400):
    n, d = x.shape
    h_dim = W0.shape[1]
    o_dim = W1.shape[1]
    nb = n // block_m

    out = pl.pallas_call(
        functools.partial(_gcn_body, nb=nb, block_m=block_m, n=n),
        grid=(2 * nb,),
        in_specs=[
            pl.BlockSpec((n, d), lambda i: (0, 0)),
            pl.BlockSpec((d, h_dim), lambda i: (0, 0)),
            pl.BlockSpec((1, h_dim), lambda i: (0, 0)),
            pl.BlockSpec((h_dim, o_dim), lambda i: (0, 0)),
            pl.BlockSpec((1, o_dim), lambda i: (0, 0)),
            pl.BlockSpec((block_m, n),
                         lambda i: (jax.lax.rem(i, nb), 0)),
        ],
        out_specs=pl.BlockSpec((block_m, o_dim),
                               lambda i: (jnp.maximum(i - nb, 0), 0)),
        out_shape=jax.ShapeDtypeStruct((n, o_dim), jnp.float32),
        scratch_shapes=[
            pltpu.VMEM((n, h_dim), jnp.float32),
            pltpu.VMEM((n, h_dim), jnp.float32),
            pltpu.VMEM((n, o_dim), jnp.float32),
            pltpu.VMEM((8, h_dim), jnp.float32),
        ],
    )(x, W0, b0.reshape(1, h_dim), W1, b1.reshape(1, o_dim), adj)
    return out


def kernel(x, adj, W0, b0, W1, b1):
    return _gcn(x, adj, W0, b0, W1, b1)


# speedup vs baseline: 1.0804x; 1.0035x over previous
import functools

import jax
import jax.numpy as jnp
from jax.experimental import pallas as pl
from jax.experimental.pallas import tpu as pltpu


def _gcn_body(x_ref, w0_ref, b0_ref, w1_ref, b1_ref, adj_ref, out_ref,
              y0_ref, h_ref, y1_ref, stats_ref, *, nb, block_m, n):
    i = pl.program_id(0)

    @pl.when(i == 0)
    def _init():
        y0_ref[...] = jnp.dot(x_ref[...], w0_ref[...],
                              preferred_element_type=jnp.float32
                              ).astype(jnp.bfloat16)
        stats_ref[...] = jnp.zeros_like(stats_ref)

    @pl.when(i < nb)
    def _layer1():
        h = jnp.dot(adj_ref[...].astype(jnp.bfloat16), y0_ref[...],
                    preferred_element_type=jnp.float32) + b0_ref[...]
        h = jnp.maximum(h, 0.0)
        h_ref[pl.ds(i * block_m, block_m), :] = h
        stats_ref[0:1, :] += jnp.sum(h, axis=0, keepdims=True)
        stats_ref[1:2, :] += jnp.sum(h * h, axis=0, keepdims=True)

    @pl.when(i == nb)
    def _bn_project():
        mean = stats_ref[0:1, :] / n
        var = stats_ref[1:2, :] / n - mean * mean
        scale = jax.lax.rsqrt(var + 1e-5)
        hn = (h_ref[...] - mean) * scale
        y1_ref[...] = jnp.dot(hn, w1_ref[...],
                              preferred_element_type=jnp.float32
                              ).astype(jnp.bfloat16)

    @pl.when(i >= nb)
    def _layer2():
        out_ref[...] = jnp.dot(adj_ref[...].astype(jnp.bfloat16), y1_ref[...],
                               preferred_element_type=jnp.float32) + b1_ref[...]


@functools.partial(jax.jit, static_argnames=("block_m",))
def _gcn(x, adj, W0, b0, W1, b1, block_m=400):
    n, d = x.shape
    h_dim = W0.shape[1]
    o_dim = W1.shape[1]
    nb = n // block_m

    out = pl.pallas_call(
        functools.partial(_gcn_body, nb=nb, block_m=block_m, n=n),
        grid=(2 * nb,),
        in_specs=[
            pl.BlockSpec((n, d), lambda i: (0, 0)),
            pl.BlockSpec((d, h_dim), lambda i: (0, 0)),
            pl.BlockSpec((1, h_dim), lambda i: (0, 0)),
            pl.BlockSpec((h_dim, o_dim), lambda i: (0, 0)),
            pl.BlockSpec((1, o_dim), lambda i: (0, 0)),
            pl.BlockSpec((block_m, n),
                         lambda i: (jax.lax.rem(i, nb), 0)),
        ],
        out_specs=pl.BlockSpec((block_m, o_dim),
                               lambda i: (jnp.maximum(i - nb, 0), 0)),
        out_shape=jax.ShapeDtypeStruct((n, o_dim), jnp.float32),
        scratch_shapes=[
            pltpu.VMEM((n, h_dim), jnp.bfloat16),
            pltpu.VMEM((n, h_dim), jnp.float32),
            pltpu.VMEM((n, o_dim), jnp.bfloat16),
            pltpu.VMEM((8, h_dim), jnp.float32),
        ],
    )(x, W0, b0.reshape(1, h_dim), W1, b1.reshape(1, o_dim), adj)
    return out


def kernel(x, adj, W0, b0, W1, b1):
    return _gcn(x, adj, W0, b0, W1, b1)
